# baseline (speedup 1.0000x reference)
import functools

import jax
import jax.numpy as jnp
from jax import lax
from jax.experimental import pallas as pl
from jax.experimental.pallas import tpu as pltpu
from jax.experimental.pallas import tpu_sc as plsc

NUM_CORES = 2
NUM_SUBCORES = 16
NW = NUM_CORES * NUM_SUBCORES
GB = 16


def _gather_stage(idx3, table, B, L, V, D):
    per_w = B // NW
    G = per_w // GB

    @functools.partial(
        pl.kernel,
        mesh=plsc.VectorSubcoreMesh(core_axis_name="c", subcore_axis_name="s"),
        out_type=jax.ShapeDtypeStruct((B * L, D), jnp.float32),
        scratch_types=[
            pltpu.VMEM((per_w, L), jnp.int32),
            pltpu.VMEM((GB * L, D), jnp.float32),
            pltpu.VMEM((GB * L, D), jnp.float32),
            pltpu.SemaphoreType.DMA,
            pltpu.SemaphoreType.DMA,
            pltpu.SemaphoreType.DMA,
            pltpu.SemaphoreType.DMA,
        ],
        compiler_params=pltpu.CompilerParams(use_tc_tiling_on_sc=False),
    )
    def gather_kernel(idx_hbm, table_hbm, out_hbm, idx_v, buf_a, buf_b,
                      gsem_a, gsem_b, osem_a, osem_b):
        wid = lax.axis_index("s") * NUM_CORES + lax.axis_index("c")
        pltpu.sync_copy(idx_hbm.at[wid], idx_v)
        out_base = wid * per_w

        bufs = (buf_a, buf_b)
        gsems = (gsem_a, gsem_b)
        osems = (osem_a, osem_b)

        def fire_gathers(g, s):
            for b in range(GB):
                pltpu.async_copy(
                    table_hbm.at[idx_v.at[g * GB + b]],
                    bufs[s].at[pl.ds(b * L, L)],
                    gsems[s],
                )

        def wait_gathers(g, s):
            for b in range(GB):
                pltpu.make_async_copy(
                    table_hbm.at[idx_v.at[g * GB + b]],
                    bufs[s].at[pl.ds(b * L, L)],
                    gsems[s],
                ).wait()

        def out_slice(g):
            return out_hbm.at[pl.ds((out_base + g * GB) * L, GB * L)]

        def fire_out(g, s):
            pltpu.async_copy(bufs[s], out_slice(g), osems[s])

        def wait_out(g, s):
            pltpu.make_async_copy(bufs[s], out_slice(g), osems[s]).wait()

        fire_gathers(0, 0)

        def pair_body(p, carry):
            for s in (0, 1):
                g = 2 * p + s
                s_next = 1 - s

                @pl.when(g >= 1)
                def _():
                    wait_out(g - 1, s_next)

                @pl.when(g < G - 1)
                def _():
                    fire_gathers(g + 1, s_next)

                wait_gathers(g, s)
                fire_out(g, s)
            return carry

        lax.fori_loop(0, G // 2, pair_body, 0)
        wait_out(G - 1, (G - 1) % 2)

    return gather_kernel(idx3, table)


RT_NB = 8


def _retile_stage(y2, B, L, D):
    per_w = B // NW
    G = per_w // RT_NB
    lines = RT_NB * L // 2

    @functools.partial(
        pl.kernel,
        mesh=plsc.VectorSubcoreMesh(core_axis_name="c", subcore_axis_name="s"),
        out_type=jax.ShapeDtypeStruct((B, L, D), jnp.float32),
        scratch_types=[
            pltpu.VMEM((lines, 2 * D), jnp.float32),
            pltpu.VMEM((lines, 2 * D), jnp.float32),
            pltpu.VMEM((RT_NB, L, D), jnp.float32),
            pltpu.SemaphoreType.DMA,
            pltpu.SemaphoreType.DMA,
            pltpu.SemaphoreType.DMA,
        ],
        compiler_params=pltpu.CompilerParams(use_tc_tiling_on_sc=True),
    )
    def retile_kernel(in_hbm, out_hbm, ina, inb, outbuf,
                      isem_a, isem_b, osem):
        wid = lax.axis_index("s") * NUM_CORES + lax.axis_index("c")
        inbufs = (ina, inb)
        isems = (isem_a, isem_b)
        lines_per_item = L // 2

        def in_slice(g):
            base = (wid * per_w + g * RT_NB) * lines_per_item
            return in_hbm.at[pl.ds(base, lines)]

        def fire_in(g, s):
            pltpu.async_copy(in_slice(g), inbufs[s], isems[s])

        def wait_in(g, s):
            pltpu.make_async_copy(in_slice(g), inbufs[s], isems[s]).wait()

        def vcopy(s):
            src = inbufs[s]

            def row_body(t, carry):
                item = t // lines_per_item
                r0 = 2 * (t % lines_per_item)
                for j in range(8):
                    v = src[t, pl.ds(j * 16, 16)]
                    outbuf[item, r0 + j // 4, pl.ds((j % 4) * 16, 16)] = v
                return carry

            lax.fori_loop(0, lines, row_body, 0)

        def out_slice(g):
            return out_hbm.at[pl.ds(wid * per_w + g * RT_NB, RT_NB)]

        def fire_out(g):
            pltpu.async_copy(outbuf, out_slice(g), osem)

        def wait_out(g):
            pltpu.make_async_copy(outbuf, out_slice(g), osem).wait()

        fire_in(0, 0)

        def pair_body(p, carry):
            for s in (0, 1):
                g = 2 * p + s
                s_next = 1 - s

                @pl.when(g < G - 1)
                def _():
                    fire_in(g + 1, s_next)

                wait_in(g, s)

                @pl.when(g >= 1)
                def _():
                    wait_out(g - 1)

                vcopy(s)
                fire_out(g)
            return carry

        lax.fori_loop(0, G // 2, pair_body, 0)
        wait_out(G - 1)

    return retile_kernel(y2)


def kernel(indices, table):
    B, L = indices.shape
    V, D = table.shape
    per_w = B // NW
    assert B % NW == 0 and per_w % GB == 0 and (per_w // GB) % 2 == 0

    idx3 = indices.reshape(NW, per_w, L)
    y = _gather_stage(idx3, table, B, L, V, D)
    y2 = y.reshape(B * L // 2, 2 * D)
    return _retile_stage(y2, B, L, D)


# speedup vs baseline: 1.2247x; 1.2247x over previous
import functools

import jax
import jax.numpy as jnp
from jax import lax
from jax.experimental import pallas as pl
from jax.experimental.pallas import tpu as pltpu
from jax.experimental.pallas import tpu_sc as plsc

NUM_CORES = 2
NUM_SUBCORES = 16
NW = NUM_CORES * NUM_SUBCORES
GB = 8


def kernel(indices, table):
    B, L = indices.shape
    V, D = table.shape
    per_w = B // NW
    G = per_w // GB
    assert B % NW == 0 and per_w % GB == 0 and G % 2 == 0

    idx3 = indices.reshape(NW, per_w, L)

    @functools.partial(
        pl.kernel,
        mesh=plsc.VectorSubcoreMesh(core_axis_name="c", subcore_axis_name="s"),
        out_type=jax.ShapeDtypeStruct((B, L, D), jnp.float32),
        scratch_types=[
            pltpu.VMEM((per_w, L), jnp.int32),
            pltpu.VMEM((GB, L, D), jnp.float32),
            pltpu.VMEM((GB, L, D), jnp.float32),
            pltpu.SemaphoreType.DMA,
            pltpu.SemaphoreType.DMA,
            pltpu.SemaphoreType.DMA,
            pltpu.SemaphoreType.DMA,
        ],
        compiler_params=pltpu.CompilerParams(use_tc_tiling_on_sc=False),
    )
    def gather_kernel(idx_hbm, table_hbm, out_hbm, idx_v, buf_a, buf_b,
                      gsem_a, gsem_b, osem_a, osem_b):
        wid = lax.axis_index("s") * NUM_CORES + lax.axis_index("c")
        pltpu.sync_copy(idx_hbm.at[wid], idx_v)
        out_base = wid * per_w

        bufs = (buf_a, buf_b)
        gsems = (gsem_a, gsem_b)
        osems = (osem_a, osem_b)

        def fire_gathers(g, s):
            for b in range(GB):
                pltpu.async_copy(
                    table_hbm.at[idx_v.at[g * GB + b]],
                    bufs[s].at[b],
                    gsems[s],
                )

        def wait_gathers(g, s):
            for b in range(GB):
                pltpu.make_async_copy(
                    table_hbm.at[idx_v.at[g * GB + b]],
                    bufs[s].at[b],
                    gsems[s],
                ).wait()

        def out_slice(g):
            return out_hbm.at[pl.ds(out_base + g * GB, GB)]

        def fire_out(g, s):
            pltpu.async_copy(bufs[s], out_slice(g), osems[s])

        def wait_out(g, s):
            pltpu.make_async_copy(bufs[s], out_slice(g), osems[s]).wait()

        fire_gathers(0, 0)

        def pair_body(p, carry):
            for s in (0, 1):
                g = 2 * p + s
                s_next = 1 - s

                @pl.when(g >= 1)
                def _():
                    wait_out(g - 1, s_next)

                @pl.when(g < G - 1)
                def _():
                    fire_gathers(g + 1, s_next)

                wait_gathers(g, s)
                fire_out(g, s)
            return carry

        lax.fori_loop(0, G // 2, pair_body, 0)
        wait_out(G - 1, (G - 1) % 2)

    return gather_kernel(idx3, table)
